# baseline (speedup 1.0000x reference)
import functools

import jax
import jax.numpy as jnp
from jax import lax
from jax.experimental import pallas as pl
from jax.experimental.pallas import tpu as pltpu
from jax.experimental.pallas import tpu_sc as plsc

P = 1000
L = 4096
B = 16
CH = 32
V = 200019
QUARTER = 50176
NKQ = QUARTER // 1024
TRASH = B * QUARTER
MAXM = (B * (L // CH) + 15) // 16
K0 = 199680
KTAIL = V - K0


def _sc_body(x_hbm, n_hbm, m_hbm,
             n_vmem, idxstage_v, sidx_v, ones_v, zeros_v, bounce_v, spmem, sem, sem2):
    c = lax.axis_index("c")
    s = lax.axis_index("s")

    pltpu.sync_copy(n_hbm, n_vmem.at[pl.ds(0, 16)])
    ns = [n_vmem[pl.ds(i, 16)][0] for i in range(B)]
    cum = [jnp.int32(0)]
    for i in range(B):
        cum.append(cum[-1] + lax.div(ns[i] + (CH - 1), CH))
    total = cum[B]
    m = lax.div(jnp.maximum(total - s + 15, 0), 16)

    def chunk_info(t):
        g = s + 16 * t
        b = jnp.int32(0)
        for i in range(1, B):
            b = b + (g >= cum[i]).astype(jnp.int32)
        cb = jnp.int32(0)
        nb = jnp.int32(0)
        for i in range(B):
            is_i = (b == i).astype(jnp.int32)
            cb = cb + is_i * cum[i]
            nb = nb + is_i * ns[i]
        start = (g - cb) * CH
        valid = jnp.minimum(nb - start, CH)
        return b, start, valid

    def stage(t, c2):
        b, start, _ = chunk_info(t)
        pltpu.async_copy(x_hbm.at[b, pl.ds(start, CH)],
                         idxstage_v.at[t, pl.ds(0, CH)], sem)
        return c2
    lax.fori_loop(0, m, stage, 0)

    zv = jnp.zeros((16,), jnp.float32)

    def zz(i, c2):
        zeros_v[pl.ds(i * 16, 16)] = zv
        return c2
    lax.fori_loop(0, 512, zz, 0)
    for g in range(8):
        ones_v[pl.ds(g * 16, 16)] = jnp.ones((16,), jnp.float32)

    def stage_drain(t, c2):
        pltpu.make_async_copy(x_hbm.at[0, pl.ds(0, CH)],
                              idxstage_v.at[0, pl.ds(0, CH)], sem).wait()
        return c2
    lax.fori_loop(0, m, stage_drain, 0)

    lane = lax.iota(jnp.int32, 16)
    sbase = s * QUARTER
    tot_e = m * CH
    nstream = lax.div(tot_e + 127, 128)
    trash_vec = jnp.zeros((16,), jnp.int32) + TRASH

    for r in range(2):
        q = c * 2 + r
        lo = q * QUARTER

        for i in range(6):
            pltpu.async_copy(zeros_v.at[pl.ds(0, 8192)],
                             spmem.at[pl.ds(sbase + i * 8192, 8192)], sem2)
        pltpu.async_copy(zeros_v.at[pl.ds(0, 1024)],
                         spmem.at[pl.ds(sbase + 49152, 1024)], sem2)
        for i in range(6):
            pltpu.make_async_copy(zeros_v.at[pl.ds(0, 8192)],
                                  spmem.at[pl.ds(0, 8192)], sem2).wait()
        pltpu.make_async_copy(zeros_v.at[pl.ds(0, 1024)],
                              spmem.at[pl.ds(0, 1024)], sem2).wait()

        def xform(t, c2):
            b, _, valid = chunk_info(t)
            for g in range(CH // 16):
                tok = idxstage_v[t, pl.ds(g * 16, 16)]
                keep = ((tok >= lo) & (tok < lo + QUARTER)
                        & ((g * 16 + lane) < valid))
                off = jnp.where(keep, b * QUARTER + tok - lo, TRASH)
                e = t * CH + g * 16
                sidx_v[lax.div(e, 128), 0, pl.ds(lax.rem(e, 128), 16)] = off
            return c2
        lax.fori_loop(0, m, xform, 0)

        def pad(pg, c2):
            e = tot_e + pg * 16
            sidx_v[lax.div(e, 128), 0,
                   pl.ds(lax.rem(e, 128), 16)] = trash_vec
            return c2
        lax.fori_loop(0, lax.div(nstream * 128 - tot_e, 16), pad, 0)
        plsc.subcore_barrier()

        def scat(k, c2):
            pltpu.async_copy(ones_v.at[pl.ds(0, 128)],
                             spmem.at[sidx_v.at[k, 0]], sem2, add=True)
            return c2
        lax.fori_loop(0, nstream, scat, 0)

        def scat_drain(k, c2):
            pltpu.make_async_copy(ones_v.at[pl.ds(0, 128)],
                                  spmem.at[pl.ds(0, 128)], sem2).wait()
            return c2
        lax.fori_loop(0, nstream, scat_drain, 0)
        plsc.subcore_barrier()

        obase = (q * B + s) * QUARTER
        for i in range(6):
            h = (i % 2) * 8192
            if i >= 2:
                pltpu.make_async_copy(bounce_v.at[pl.ds(0, 8192)],
                                      m_hbm.at[pl.ds(0, 8192)], sem2).wait()
            pltpu.sync_copy(spmem.at[pl.ds(sbase + i * 8192, 8192)],
                            bounce_v.at[pl.ds(h, 8192)])
            pltpu.async_copy(bounce_v.at[pl.ds(h, 8192)],
                             m_hbm.at[pl.ds(obase + i * 8192, 8192)], sem2)
        pltpu.make_async_copy(bounce_v.at[pl.ds(0, 8192)],
                              m_hbm.at[pl.ds(0, 8192)], sem2).wait()
        pltpu.sync_copy(spmem.at[pl.ds(sbase + 49152, 1024)],
                        bounce_v.at[pl.ds(0, 1024)])
        pltpu.async_copy(bounce_v.at[pl.ds(0, 1024)],
                         m_hbm.at[pl.ds(obase + 49152, 1024)], sem2)
        pltpu.make_async_copy(bounce_v.at[pl.ds(0, 8192)],
                              m_hbm.at[pl.ds(0, 8192)], sem2).wait()
        pltpu.make_async_copy(bounce_v.at[pl.ds(0, 1024)],
                              m_hbm.at[pl.ds(0, 1024)], sem2).wait()
        if r == 0:
            plsc.subcore_barrier()


def _scatter_sc(X, N):
    mesh = plsc.VectorSubcoreMesh(core_axis_name="c", subcore_axis_name="s")
    f = pl.kernel(
        _sc_body,
        out_type=jax.ShapeDtypeStruct((4 * B * QUARTER,), jnp.float32),
        mesh=mesh,
        scratch_types=[
            pltpu.VMEM((32,), jnp.int32),
            pltpu.VMEM((MAXM, CH + 16), jnp.int32),
            pltpu.VMEM((MAXM * CH // 128, 1, 128), jnp.int32),
            pltpu.VMEM((128,), jnp.float32),
            pltpu.VMEM((8192,), jnp.float32),
            pltpu.VMEM((16384,), jnp.float32),
            pltpu.VMEM_SHARED((B * QUARTER + 16,), jnp.float32),
            pltpu.SemaphoreType.DMA,
            pltpu.SemaphoreType.DMA,
        ],
    )
    return f(X, N)


def _mm_body(m_ref, t_ref, o_ref):
    @pl.when(pl.program_id(0) == 0)
    def _():
        o_ref[...] = jnp.zeros_like(o_ref)
    o_ref[...] += lax.dot_general(
        m_ref[...], t_ref[...], (((1,), (1,)), ((), ())),
        preferred_element_type=jnp.float32)


def _mm(M64, tableT):
    return pl.pallas_call(
        _mm_body,
        grid=(K0 // 1024,),
        in_specs=[
            pl.BlockSpec((B, 1024), lambda k: (k // NKQ, k % NKQ)),
            pl.BlockSpec((P, 1024), lambda k: (0, k)),
        ],
        out_specs=pl.BlockSpec((B, P), lambda k: (0, 0)),
        out_shape=jax.ShapeDtypeStruct((B, P), jnp.float32),
    )(M64, tableT)


def _tail_body(s1_ref, mt_ref, tt_ref, n_ref, gamma_ref, beta_ref,
               w_ref, bias_ref, out_ref):
    s = s1_ref[...] + lax.dot_general(
        mt_ref[...], tt_ref[...], (((1,), (1,)), ((), ())),
        preferred_element_type=jnp.float32)
    nf = n_ref[...].astype(jnp.float32)
    x = s / nf
    mean = jnp.mean(x, axis=1, keepdims=True)
    xc = x - mean
    var = jnp.mean(xc * xc, axis=1, keepdims=True)
    xn = xc * lax.rsqrt(var + 1e-5)
    xn = xn * gamma_ref[...] + beta_ref[...]
    z = jnp.sum(xn * w_ref[...], axis=1, keepdims=True) + bias_ref[...]
    out_ref[...] = jax.nn.sigmoid(z)


def _tail_tc(s1, mt, tt, N, gamma, beta, W, b):
    return pl.pallas_call(
        _tail_body,
        out_shape=jax.ShapeDtypeStruct((B, 1), jnp.float32),
    )(s1, mt, tt, N.reshape(B, 1), gamma.reshape(1, P), beta.reshape(1, P),
      W.reshape(1, P), b.reshape(1, 1))


@jax.jit
def kernel(X, N, table, gamma, beta, W, b):
    X = X.astype(jnp.int32)
    N = N.astype(jnp.int32)
    tableT = table.T
    M64 = _scatter_sc(X, N).reshape(4 * B, QUARTER)
    s1 = _mm(M64, tableT)
    mt = lax.slice(M64, (3 * B, K0 - 3 * QUARTER), (4 * B, V - 3 * QUARTER))
    tt = lax.slice(tableT, (0, K0), (P, V))
    return _tail_tc(s1, mt, tt, N, gamma, beta, W, b).reshape(B)


# speedup vs baseline: 1.0951x; 1.0951x over previous
import functools

import jax
import jax.numpy as jnp
from jax import lax
from jax.experimental import pallas as pl
from jax.experimental.pallas import tpu as pltpu
from jax.experimental.pallas import tpu_sc as plsc

P = 1000
L = 4096
B = 16
CH = 32
V = 200019
HALF = 100352
NKH = HALF // 1024
TRASH = B * HALF
MAXM = (B * (L // CH) + 15) // 16
K0 = 199680
KTAIL = V - K0


def _sc_body(x_hbm, n_hbm, m_hbm,
             n_vmem, idxstage_v, sidx_v, ones_v, zeros_v, spmem, sem, sem2):
    c = lax.axis_index("c")
    s = lax.axis_index("s")

    pltpu.sync_copy(n_hbm, n_vmem.at[pl.ds(0, 16)])
    ns = [n_vmem[pl.ds(i, 16)][0] for i in range(B)]
    cum = [jnp.int32(0)]
    for i in range(B):
        cum.append(cum[-1] + lax.div(ns[i] + (CH - 1), CH))
    total = cum[B]
    m = lax.div(jnp.maximum(total - s + 15, 0), 16)

    def chunk_info(t):
        g = s + 16 * t
        b = jnp.int32(0)
        for i in range(1, B):
            b = b + (g >= cum[i]).astype(jnp.int32)
        cb = jnp.int32(0)
        nb = jnp.int32(0)
        for i in range(B):
            is_i = (b == i).astype(jnp.int32)
            cb = cb + is_i * cum[i]
            nb = nb + is_i * ns[i]
        start = (g - cb) * CH
        valid = jnp.minimum(nb - start, CH)
        return b, start, valid

    def stage(t, c2):
        b, start, _ = chunk_info(t)
        pltpu.async_copy(x_hbm.at[b, pl.ds(start, CH)],
                         idxstage_v.at[t, pl.ds(0, CH)], sem)
        return c2
    lax.fori_loop(0, m, stage, 0)

    zv = jnp.zeros((16,), jnp.float32)

    def zz(i, c2):
        zeros_v[pl.ds(i * 16, 16)] = zv
        return c2
    lax.fori_loop(0, 512, zz, 0)
    for g in range(8):
        ones_v[pl.ds(g * 16, 16)] = jnp.ones((16,), jnp.float32)

    def stage_drain(t, c2):
        pltpu.make_async_copy(x_hbm.at[0, pl.ds(0, CH)],
                              idxstage_v.at[0, pl.ds(0, CH)], sem).wait()
        return c2
    lax.fori_loop(0, m, stage_drain, 0)

    lane = lax.iota(jnp.int32, 16)
    sbase = s * HALF
    lo = c * HALF
    tot_e = m * CH
    nstream = lax.div(tot_e + 127, 128)
    trash_vec = jnp.zeros((16,), jnp.int32) + TRASH

    for i in range(12):
        pltpu.async_copy(zeros_v.at[pl.ds(0, 8192)],
                         spmem.at[pl.ds(sbase + i * 8192, 8192)], sem2)
    pltpu.async_copy(zeros_v.at[pl.ds(0, 2048)],
                     spmem.at[pl.ds(sbase + 98304, 2048)], sem2)
    for i in range(12):
        pltpu.make_async_copy(zeros_v.at[pl.ds(0, 8192)],
                              spmem.at[pl.ds(0, 8192)], sem2).wait()
    pltpu.make_async_copy(zeros_v.at[pl.ds(0, 2048)],
                          spmem.at[pl.ds(0, 2048)], sem2).wait()

    def xform(t, c2):
        b, _, valid = chunk_info(t)
        for g in range(CH // 16):
            tok = idxstage_v[t, pl.ds(g * 16, 16)]
            keep = ((tok >= lo) & (tok < lo + HALF)
                    & ((g * 16 + lane) < valid))
            off = jnp.where(keep, b * HALF + tok - lo, TRASH)
            e = t * CH + g * 16
            sidx_v[lax.div(e, 1024), lax.div(lax.rem(e, 1024), 128),
                   pl.ds(lax.rem(e, 128), 16)] = off
        return c2
    lax.fori_loop(0, m, xform, 0)

    def pad(pg, c2):
        e = tot_e + pg * 16
        sidx_v[lax.div(e, 1024), lax.div(lax.rem(e, 1024), 128),
               pl.ds(lax.rem(e, 128), 16)] = trash_vec
        return c2
    lax.fori_loop(0, lax.div(nstream * 128 - tot_e, 16), pad, 0)
    plsc.subcore_barrier()

    def scat(k, c2):
        pltpu.async_copy(ones_v.at[pl.ds(0, 128)],
                         spmem.at[sidx_v.at[lax.div(k, 8), lax.rem(k, 8)]],
                         sem2, add=True)
        return c2
    lax.fori_loop(0, nstream, scat, 0)

    def scat_drain(k, c2):
        pltpu.make_async_copy(ones_v.at[pl.ds(0, 128)],
                              spmem.at[pl.ds(0, 128)], sem2).wait()
        return c2
    lax.fori_loop(0, nstream, scat_drain, 0)
    plsc.subcore_barrier()

    obase = (c * B + s) * HALF
    for i in range(24):
        h = (i % 2) * 4096
        if i >= 2:
            pltpu.make_async_copy(zeros_v.at[pl.ds(0, 4096)],
                                  m_hbm.at[pl.ds(0, 4096)], sem2).wait()
        pltpu.sync_copy(spmem.at[pl.ds(sbase + i * 4096, 4096)],
                        zeros_v.at[pl.ds(h, 4096)])
        pltpu.async_copy(zeros_v.at[pl.ds(h, 4096)],
                         m_hbm.at[pl.ds(obase + i * 4096, 4096)], sem2)
    pltpu.make_async_copy(zeros_v.at[pl.ds(0, 4096)],
                          m_hbm.at[pl.ds(0, 4096)], sem2).wait()
    pltpu.sync_copy(spmem.at[pl.ds(sbase + 98304, 2048)],
                    zeros_v.at[pl.ds(0, 2048)])
    pltpu.async_copy(zeros_v.at[pl.ds(0, 2048)],
                     m_hbm.at[pl.ds(obase + 98304, 2048)], sem2)
    pltpu.make_async_copy(zeros_v.at[pl.ds(0, 4096)],
                          m_hbm.at[pl.ds(0, 4096)], sem2).wait()
    pltpu.make_async_copy(zeros_v.at[pl.ds(0, 2048)],
                          m_hbm.at[pl.ds(0, 2048)], sem2).wait()


def _scatter_sc(X, N):
    mesh = plsc.VectorSubcoreMesh(core_axis_name="c", subcore_axis_name="s")
    f = pl.kernel(
        _sc_body,
        out_type=jax.ShapeDtypeStruct((2 * B * HALF,), jnp.float32),
        mesh=mesh,
        scratch_types=[
            pltpu.VMEM((32,), jnp.int32),
            pltpu.VMEM((MAXM, CH + 16), jnp.int32),
            pltpu.VMEM((MAXM * CH // 1024, 8, 128), jnp.int32),
            pltpu.VMEM((128,), jnp.float32),
            pltpu.VMEM((8192,), jnp.float32),
            pltpu.VMEM_SHARED((B * HALF + 16,), jnp.float32),
            pltpu.SemaphoreType.DMA,
            pltpu.SemaphoreType.DMA,
        ],
    )
    return f(X, N)


def _mm_body(m_ref, t_ref, o_ref):
    @pl.when(pl.program_id(0) == 0)
    def _():
        o_ref[...] = jnp.zeros_like(o_ref)
    o_ref[...] += lax.dot_general(
        m_ref[...], t_ref[...], (((1,), (1,)), ((), ())),
        preferred_element_type=jnp.float32)


def _mm(M32, tableT):
    return pl.pallas_call(
        _mm_body,
        grid=(K0 // 1024,),
        in_specs=[
            pl.BlockSpec((B, 1024), lambda k: (k // NKH, k % NKH)),
            pl.BlockSpec((P, 1024), lambda k: (0, k)),
        ],
        out_specs=pl.BlockSpec((B, P), lambda k: (0, 0)),
        out_shape=jax.ShapeDtypeStruct((B, P), jnp.float32),
    )(M32, tableT)


def _tail_body(s1_ref, mt_ref, tt_ref, n_ref, gamma_ref, beta_ref,
               w_ref, bias_ref, out_ref):
    s = s1_ref[...] + lax.dot_general(
        mt_ref[...], tt_ref[...], (((1,), (1,)), ((), ())),
        preferred_element_type=jnp.float32)
    nf = n_ref[...].astype(jnp.float32)
    x = s / nf
    mean = jnp.mean(x, axis=1, keepdims=True)
    xc = x - mean
    var = jnp.mean(xc * xc, axis=1, keepdims=True)
    xn = xc * lax.rsqrt(var + 1e-5)
    xn = xn * gamma_ref[...] + beta_ref[...]
    z = jnp.sum(xn * w_ref[...], axis=1, keepdims=True) + bias_ref[...]
    out_ref[...] = jax.nn.sigmoid(z)


def _tail_tc(s1, mt, tt, N, gamma, beta, W, b):
    return pl.pallas_call(
        _tail_body,
        out_shape=jax.ShapeDtypeStruct((B, 1), jnp.float32),
    )(s1, mt, tt, N.reshape(B, 1), gamma.reshape(1, P), beta.reshape(1, P),
      W.reshape(1, P), b.reshape(1, 1))


@jax.jit
def kernel(X, N, table, gamma, beta, W, b):
    X = X.astype(jnp.int32)
    N = N.astype(jnp.int32)
    tableT = table.T
    M32 = _scatter_sc(X, N).reshape(2 * B, HALF)
    s1 = _mm(M32, tableT)
    mt = lax.slice(M32, (B, K0 - HALF), (2 * B, V - HALF))
    tt = lax.slice(tableT, (0, K0), (P, V))
    return _tail_tc(s1, mt, tt, N, gamma, beta, W, b).reshape(B)
